# Initial kernel scaffold; baseline (speedup 1.0000x reference)
import jax
import jax.numpy as jnp
from jax.experimental import pallas as pl


def kernel(x, edge_index, edge_attr, Wn, bn_enc, W, b, We, be, root, gamma, beta):
    raise NotImplementedError("write your pallas kernel here")


# speedup vs baseline: 3.6696x; 3.6696x over previous
import functools

import jax
import jax.numpy as jnp
from jax import lax
from jax.experimental import pallas as pl
from jax.experimental.pallas import tpu as pltpu
from jax.experimental.pallas import tpu_sc as plsc

N = 10000
E = 320000
D = 128
ED = 16
L = 3

NC = 2
NS = 16
NW = NC * NS
EW = E // NW
C = 40
G = EW // C
SG = 50
SN = G // SG
NP = 10240
RS = NP // NS
WB = 32

_MESH = plsc.VectorSubcoreMesh(core_axis_name="c", subcore_axis_name="s")


def _wid():
    return lax.axis_index("s") * NC + lax.axis_index("c")


GD = 125
CD = 80


def _deg_body(row_hbm, out_hbm, deg_sh, row_v, ones_v, buf_v):
    cid = lax.axis_index("c")
    sid = lax.axis_index("s")
    wid = _wid()
    for i in range(CD // 16):
        ones_v[pl.ds(i * 16, 16)] = jnp.ones((16,), jnp.float32)
    for i in range(RS // 16):
        buf_v[pl.ds(i * 16, 16)] = jnp.zeros((16,), jnp.float32)
    pltpu.sync_copy(buf_v, deg_sh.at[pl.ds(sid * RS, RS)])
    plsc.subcore_barrier()
    pltpu.sync_copy(row_hbm.at[wid], row_v)

    def chunk(g, carry):
        pltpu.sync_copy(ones_v, deg_sh.at[row_v.at[g]], add=True)
        return carry

    lax.fori_loop(0, GD, chunk, 0)
    plsc.subcore_barrier()
    pltpu.sync_copy(deg_sh.at[pl.ds(sid * RS, RS)], buf_v)
    pltpu.sync_copy(buf_v, out_hbm.at[pl.ds(cid * NP + sid * RS, RS)])


_deg_call = pl.kernel(
    _deg_body,
    out_type=jax.ShapeDtypeStruct((NC * NP,), jnp.float32),
    mesh=_MESH,
    compiler_params=pltpu.CompilerParams(needs_layout_passes=False),
    scratch_types=[
        pltpu.VMEM_SHARED((NP,), jnp.float32),
        pltpu.VMEM((GD, CD), jnp.int32),
        pltpu.VMEM((CD,), jnp.float32),
        pltpu.VMEM((RS,), jnp.float32),
    ],
)


def _norm_body(dis_hbm, row_hbm, col_hbm, out_hbm, dis_v, row_v, col_v, nrm_v):
    wid = _wid()
    pltpu.sync_copy(dis_hbm, dis_v)
    pltpu.sync_copy(row_hbm.at[wid], row_v)
    pltpu.sync_copy(col_hbm.at[wid], col_v)

    def step(i, carry):
        rv = row_v[0, pl.ds(i * 16, 16)]
        cv = col_v[0, pl.ds(i * 16, 16)]
        a = plsc.load_gather(dis_v, [rv])
        bb = plsc.load_gather(dis_v, [cv])
        nrm_v[0, pl.ds(i * 16, 16)] = a * bb
        return carry

    lax.fori_loop(0, EW // 16, step, 0)
    pltpu.sync_copy(nrm_v, out_hbm.at[wid])


_norm_call = pl.kernel(
    _norm_body,
    out_type=jax.ShapeDtypeStruct((NW, 1, EW), jnp.float32),
    mesh=_MESH,
    compiler_params=pltpu.CompilerParams(needs_layout_passes=False),
    scratch_types=[
        pltpu.VMEM((N,), jnp.float32),
        pltpu.VMEM((1, EW), jnp.int32),
        pltpu.VMEM((1, EW), jnp.int32),
        pltpu.VMEM((1, EW), jnp.float32),
    ],
)


def _edge_body(hx_hbm, ee_hbm, row_hbm, col_hbm, nrm_hbm, out_hbm,
               aggr_sh, row_v, col_v, nrm_v,
               hxg0, hxg1, eev0, eev1, wbuf, sem0, sem1):
    cid = lax.axis_index("c")
    sid = lax.axis_index("s")
    wid = _wid()

    def zero_block(i, carry):
        for j in range(D // 16):
            wbuf[i, pl.ds(j * 16, 16)] = jnp.zeros((16,), jnp.float32)
        return carry

    lax.fori_loop(0, WB, zero_block, 0)
    for k in range(RS // WB):
        pltpu.sync_copy(wbuf, aggr_sh.at[pl.ds(sid * RS + k * WB, WB)])
    plsc.subcore_barrier()

    ebase = wid * EW

    def issue(s, g, hxg, eev, sem):
        d1 = pltpu.async_copy(hx_hbm.at[row_v.at[g]], hxg, sem)
        d2 = pltpu.async_copy(
            ee_hbm.at[pl.ds(ebase + (s * SG + g) * C, C)], eev, sem)
        return d1, d2

    def compute_chunk(g, hxg, eev):
        def edge(e, c2):
            e16 = jnp.zeros((16,), jnp.int32) + e
            g16 = jnp.zeros((16,), jnp.int32) + g
            nv = plsc.load_gather(nrm_v, [g16, e16])
            for j in range(D // 16):
                hv = hxg[e, pl.ds(j * 16, 16)]
                av = eev[e, pl.ds(j * 16, 16)]
                hxg[e, pl.ds(j * 16, 16)] = nv * jnp.maximum(hv + av, 0.0)
            return c2

        lax.fori_loop(0, C, edge, 0)

    def superchunk(s, carry):
        pltpu.sync_copy(row_hbm.at[wid, s], row_v)
        pltpu.sync_copy(col_hbm.at[wid, s], col_v)
        pltpu.sync_copy(nrm_hbm.at[wid, s], nrm_v)

        def pair(g2, c1):
            a = 2 * g2
            b = a + 1
            da1, da2 = issue(s, a, hxg0, eev0, sem0)
            db1, db2 = issue(s, b, hxg1, eev1, sem1)
            da1.wait()
            da2.wait()
            compute_chunk(a, hxg0, eev0)
            pltpu.sync_copy(hxg0, aggr_sh.at[col_v.at[a]], add=True)
            db1.wait()
            db2.wait()
            compute_chunk(b, hxg1, eev1)
            pltpu.sync_copy(hxg1, aggr_sh.at[col_v.at[b]], add=True)
            return c1

        lax.fori_loop(0, SG // 2, pair, 0)
        return carry

    lax.fori_loop(0, SN, superchunk, 0)
    plsc.subcore_barrier()
    for k in range(RS // WB):
        r0 = sid * RS + k * WB
        pltpu.sync_copy(aggr_sh.at[pl.ds(r0, WB)], wbuf)
        pltpu.sync_copy(wbuf, out_hbm.at[cid, pl.ds(r0, WB)])


_edge_call = pl.kernel(
    _edge_body,
    out_type=jax.ShapeDtypeStruct((NC, NP, D), jnp.float32),
    mesh=_MESH,
    compiler_params=pltpu.CompilerParams(needs_layout_passes=False),
    scratch_types=[
        pltpu.VMEM_SHARED((NP, D), jnp.float32),
        pltpu.VMEM((SG, C), jnp.int32),
        pltpu.VMEM((SG, C), jnp.int32),
        pltpu.VMEM((SG, C), jnp.float32),
        pltpu.VMEM((C, D), jnp.float32),
        pltpu.VMEM((C, D), jnp.float32),
        pltpu.VMEM((C, D), jnp.float32),
        pltpu.VMEM((C, D), jnp.float32),
        pltpu.VMEM((WB, D), jnp.float32),
        pltpu.SemaphoreType.DMA,
        pltpu.SemaphoreType.DMA,
    ],
)


def _enc_tc(x_ref, w_ref, b_ref, o_ref):
    o_ref[...] = lax.dot_general(
        x_ref[...], w_ref[...], (((1,), (1,)), ((), ())),
        preferred_element_type=jnp.float32) + b_ref[...]


def _hx_tc(h_ref, w_ref, b_ref, r_ref, dg_ref, hx_ref, st_ref):
    hx = lax.dot_general(
        h_ref[...], w_ref[...], (((1,), (1,)), ((), ())),
        preferred_element_type=jnp.float32) + b_ref[...]
    hx_ref[...] = hx
    st_ref[...] = jnp.maximum(hx + r_ref[...], 0.0) / dg_ref[...]


BE = 8000


def _ee_tc(ea_ref, w_ref, b_ref, o_ref):
    o_ref[...] = lax.dot_general(
        ea_ref[...], w_ref[...], (((1,), (1,)), ((), ())),
        preferred_element_type=jnp.float32) + b_ref[...]


def _post_tc(a0_ref, a1_ref, st_ref, g_ref, bt_ref, o_ref, *, relu):
    t = a0_ref[...] + a1_ref[...] + st_ref[...]
    mean = jnp.mean(t, axis=0, keepdims=True)
    var = jnp.mean((t - mean) ** 2, axis=0, keepdims=True)
    y = (t - mean) / jnp.sqrt(var + 1e-5) * g_ref[...] + bt_ref[...]
    o_ref[...] = jnp.maximum(y, 0.0) if relu else y


def _enc_call(x, Wn, bn):
    return pl.pallas_call(
        _enc_tc,
        out_shape=jax.ShapeDtypeStruct((N, D), jnp.float32),
    )(x, Wn, bn)


def _hx_call(h, Wl, bl, rl, degc):
    return pl.pallas_call(
        _hx_tc,
        out_shape=(jax.ShapeDtypeStruct((N, D), jnp.float32),
                   jax.ShapeDtypeStruct((N, D), jnp.float32)),
    )(h, Wl, bl, rl, degc)


def _ee_call(edge_attr, Wel, bel):
    return pl.pallas_call(
        _ee_tc,
        grid=(E // BE,),
        in_specs=[pl.BlockSpec((BE, ED), lambda i: (i, 0)),
                  pl.BlockSpec((D, ED), lambda i: (0, 0)),
                  pl.BlockSpec((1, D), lambda i: (0, 0))],
        out_specs=pl.BlockSpec((BE, D), lambda i: (i, 0)),
        out_shape=jax.ShapeDtypeStruct((E, D), jnp.float32),
    )(edge_attr, Wel, bel)


def _post_call(a0, a1, st, gl, btl, relu):
    return pl.pallas_call(
        functools.partial(_post_tc, relu=relu),
        out_shape=jax.ShapeDtypeStruct((N, D), jnp.float32),
    )(a0, a1, st, gl, btl)


def kernel(x, edge_index, edge_attr, Wn, bn_enc, W, b, We, be, root, gamma, beta):
    row = edge_index[0]
    col = edge_index[1]
    row2 = row.reshape(NW, SN, SG, C)
    col2 = col.reshape(NW, SN, SG, C)

    h = _enc_call(x, Wn, bn_enc.reshape(1, D))

    degp = _deg_call(row.reshape(NW, GD, CD)).reshape(NC, NP)
    deg = degp[0, :N] + degp[1, :N] + 1.0
    dis = deg ** -0.5
    nrm2 = _norm_call(dis, row.reshape(NW, 1, EW), col.reshape(NW, 1, EW))
    degc = deg.reshape(N, 1)

    for l in range(L):
        hx, st = _hx_call(h, W[l], b[l].reshape(1, D), root[l].reshape(1, D), degc)
        ee = _ee_call(edge_attr, We[l], be[l].reshape(1, D))
        aggrp = _edge_call(hx, ee, row2, col2,
                           nrm2.reshape(NW, SN, SG, C))
        h = _post_call(aggrp[0, :N], aggrp[1, :N], st,
                       gamma[l].reshape(1, D), beta[l].reshape(1, D),
                       relu=(l < L - 1))
    return h


# speedup vs baseline: 4.2446x; 1.1567x over previous
import functools

import jax
import jax.numpy as jnp
from jax import lax
from jax.experimental import pallas as pl
from jax.experimental.pallas import tpu as pltpu
from jax.experimental.pallas import tpu_sc as plsc

N = 10000
E = 320000
D = 128
ED = 16
L = 3

NC = 2
NS = 16
NW = NC * NS
EW = E // NW
C = 40
G = EW // C
SG = 50
SN = G // SG
NP = 10240
RS = NP // NS
WB = 32

_MESH = plsc.VectorSubcoreMesh(core_axis_name="c", subcore_axis_name="s")


def _wid():
    return lax.axis_index("s") * NC + lax.axis_index("c")


GD = 125
CD = 80


def _deg_body(row_hbm, out_hbm, deg_sh, row_v, ones_v, buf_v):
    cid = lax.axis_index("c")
    sid = lax.axis_index("s")
    wid = _wid()
    for i in range(CD // 16):
        ones_v[pl.ds(i * 16, 16)] = jnp.ones((16,), jnp.float32)
    for i in range(RS // 16):
        buf_v[pl.ds(i * 16, 16)] = jnp.zeros((16,), jnp.float32)
    pltpu.sync_copy(buf_v, deg_sh.at[pl.ds(sid * RS, RS)])
    plsc.subcore_barrier()
    pltpu.sync_copy(row_hbm.at[wid], row_v)

    def chunk(g, carry):
        pltpu.sync_copy(ones_v, deg_sh.at[row_v.at[g]], add=True)
        return carry

    lax.fori_loop(0, GD, chunk, 0)
    plsc.subcore_barrier()
    pltpu.sync_copy(deg_sh.at[pl.ds(sid * RS, RS)], buf_v)
    pltpu.sync_copy(buf_v, out_hbm.at[pl.ds(cid * NP + sid * RS, RS)])


_deg_call = pl.kernel(
    _deg_body,
    out_type=jax.ShapeDtypeStruct((NC * NP,), jnp.float32),
    mesh=_MESH,
    compiler_params=pltpu.CompilerParams(needs_layout_passes=False),
    scratch_types=[
        pltpu.VMEM_SHARED((NP,), jnp.float32),
        pltpu.VMEM((GD, CD), jnp.int32),
        pltpu.VMEM((CD,), jnp.float32),
        pltpu.VMEM((RS,), jnp.float32),
    ],
)


def _norm_body(dis_hbm, row_hbm, col_hbm, out_hbm, dis_v, row_v, col_v, nrm_v):
    wid = _wid()
    pltpu.sync_copy(dis_hbm, dis_v)
    pltpu.sync_copy(row_hbm.at[wid], row_v)
    pltpu.sync_copy(col_hbm.at[wid], col_v)

    def step(i, carry):
        rv = row_v[0, pl.ds(i * 16, 16)]
        cv = col_v[0, pl.ds(i * 16, 16)]
        a = plsc.load_gather(dis_v, [rv])
        bb = plsc.load_gather(dis_v, [cv])
        nrm_v[0, pl.ds(i * 16, 16)] = a * bb
        return carry

    lax.fori_loop(0, EW // 16, step, 0)
    pltpu.sync_copy(nrm_v, out_hbm.at[wid])


_norm_call = pl.kernel(
    _norm_body,
    out_type=jax.ShapeDtypeStruct((NW, 1, EW), jnp.float32),
    mesh=_MESH,
    compiler_params=pltpu.CompilerParams(needs_layout_passes=False),
    scratch_types=[
        pltpu.VMEM((N,), jnp.float32),
        pltpu.VMEM((1, EW), jnp.int32),
        pltpu.VMEM((1, EW), jnp.int32),
        pltpu.VMEM((1, EW), jnp.float32),
    ],
)


def _edge_body(hx_hbm, ee_hbm, row_hbm, col_hbm, nrm_hbm, out_hbm,
               aggr_sh, row_v, col_v, nrm_v,
               hxg0, hxg1, eev0, eev1, msg, sem0, sem1):
    cid = lax.axis_index("c")
    sid = lax.axis_index("s")
    wid = _wid()

    def zero_block(i, carry):
        for j in range(D // 16):
            msg[i, pl.ds(j * 16, 16)] = jnp.zeros((16,), jnp.float32)
        return carry

    lax.fori_loop(0, C, zero_block, 0)
    for k in range(RS // C):
        pltpu.sync_copy(msg, aggr_sh.at[pl.ds(sid * RS + k * C, C)])
    plsc.subcore_barrier()

    ebase = wid * EW

    def issue(s, g, hxg, eev, sem):
        pltpu.async_copy(hx_hbm.at[row_v.at[g]], hxg, sem)
        pltpu.async_copy(ee_hbm.at[pl.ds(ebase + (s * SG + g) * C, C)], eev, sem)

    def wait(s, g, hxg, eev, sem):
        pltpu.make_async_copy(hx_hbm.at[row_v.at[g]], hxg, sem).wait()
        pltpu.make_async_copy(
            ee_hbm.at[pl.ds(ebase + (s * SG + g) * C, C)], eev, sem).wait()

    def compute_chunk(g, hxg, eev):
        g16 = jnp.zeros((16,), jnp.int32) + g

        def edge(e2, c2):
            for u in range(2):
                e = e2 * 2 + u
                e16 = jnp.zeros((16,), jnp.int32) + e
                nv = plsc.load_gather(nrm_v, [g16, e16])
                for j in range(D // 16):
                    hv = hxg[e, pl.ds(j * 16, 16)]
                    av = eev[e, pl.ds(j * 16, 16)]
                    msg[e, pl.ds(j * 16, 16)] = nv * jnp.maximum(hv + av, 0.0)
            return c2

        lax.fori_loop(0, C // 2, edge, 0)

    def superchunk(s, carry):
        pltpu.sync_copy(row_hbm.at[wid, s], row_v)
        pltpu.sync_copy(col_hbm.at[wid, s], col_v)
        pltpu.sync_copy(nrm_hbm.at[wid, s], nrm_v)
        issue(s, 0, hxg0, eev0, sem0)

        def pair(g2, c1):
            a = 2 * g2
            b = a + 1
            issue(s, b, hxg1, eev1, sem1)
            wait(s, a, hxg0, eev0, sem0)
            compute_chunk(a, hxg0, eev0)

            @pl.when(g2 < SG // 2 - 1)
            def _():
                issue(s, a + 2, hxg0, eev0, sem0)

            pltpu.sync_copy(msg, aggr_sh.at[col_v.at[a]], add=True)
            wait(s, b, hxg1, eev1, sem1)
            compute_chunk(b, hxg1, eev1)
            pltpu.sync_copy(msg, aggr_sh.at[col_v.at[b]], add=True)
            return c1

        lax.fori_loop(0, SG // 2, pair, 0)
        return carry

    lax.fori_loop(0, SN, superchunk, 0)
    plsc.subcore_barrier()
    for k in range(RS // C):
        r0 = sid * RS + k * C
        pltpu.sync_copy(aggr_sh.at[pl.ds(r0, C)], msg)
        pltpu.sync_copy(msg, out_hbm.at[cid, pl.ds(r0, C)])


_edge_call = pl.kernel(
    _edge_body,
    out_type=jax.ShapeDtypeStruct((NC, NP, D), jnp.float32),
    mesh=_MESH,
    compiler_params=pltpu.CompilerParams(needs_layout_passes=False),
    scratch_types=[
        pltpu.VMEM_SHARED((NP, D), jnp.float32),
        pltpu.VMEM((SG, C), jnp.int32),
        pltpu.VMEM((SG, C), jnp.int32),
        pltpu.VMEM((SG, C), jnp.float32),
        pltpu.VMEM((C, D), jnp.float32),
        pltpu.VMEM((C, D), jnp.float32),
        pltpu.VMEM((C, D), jnp.float32),
        pltpu.VMEM((C, D), jnp.float32),
        pltpu.VMEM((C, D), jnp.float32),
        pltpu.SemaphoreType.DMA,
        pltpu.SemaphoreType.DMA,
    ],
)


def _enc_tc(x_ref, w_ref, b_ref, o_ref):
    o_ref[...] = lax.dot_general(
        x_ref[...], w_ref[...], (((1,), (1,)), ((), ())),
        preferred_element_type=jnp.float32) + b_ref[...]


def _hx_tc(h_ref, w_ref, b_ref, r_ref, dg_ref, hx_ref, st_ref):
    hx = lax.dot_general(
        h_ref[...], w_ref[...], (((1,), (1,)), ((), ())),
        preferred_element_type=jnp.float32) + b_ref[...]
    hx_ref[...] = hx
    st_ref[...] = jnp.maximum(hx + r_ref[...], 0.0) / dg_ref[...]


BE = 8000


def _ee_tc(ea_ref, w_ref, b_ref, o_ref):
    o_ref[...] = lax.dot_general(
        ea_ref[...], w_ref[...], (((1,), (1,)), ((), ())),
        preferred_element_type=jnp.float32) + b_ref[...]


def _post_tc(a0_ref, a1_ref, st_ref, g_ref, bt_ref, o_ref, *, relu):
    t = a0_ref[...] + a1_ref[...] + st_ref[...]
    mean = jnp.mean(t, axis=0, keepdims=True)
    var = jnp.mean((t - mean) ** 2, axis=0, keepdims=True)
    y = (t - mean) / jnp.sqrt(var + 1e-5) * g_ref[...] + bt_ref[...]
    o_ref[...] = jnp.maximum(y, 0.0) if relu else y


def _enc_call(x, Wn, bn):
    return pl.pallas_call(
        _enc_tc,
        out_shape=jax.ShapeDtypeStruct((N, D), jnp.float32),
    )(x, Wn, bn)


def _hx_call(h, Wl, bl, rl, degc):
    return pl.pallas_call(
        _hx_tc,
        out_shape=(jax.ShapeDtypeStruct((N, D), jnp.float32),
                   jax.ShapeDtypeStruct((N, D), jnp.float32)),
    )(h, Wl, bl, rl, degc)


def _ee_call(edge_attr, Wel, bel):
    return pl.pallas_call(
        _ee_tc,
        grid=(E // BE,),
        in_specs=[pl.BlockSpec((BE, ED), lambda i: (i, 0)),
                  pl.BlockSpec((D, ED), lambda i: (0, 0)),
                  pl.BlockSpec((1, D), lambda i: (0, 0))],
        out_specs=pl.BlockSpec((BE, D), lambda i: (i, 0)),
        out_shape=jax.ShapeDtypeStruct((E, D), jnp.float32),
    )(edge_attr, Wel, bel)


def _post_call(a0, a1, st, gl, btl, relu):
    return pl.pallas_call(
        functools.partial(_post_tc, relu=relu),
        out_shape=jax.ShapeDtypeStruct((N, D), jnp.float32),
    )(a0, a1, st, gl, btl)


def kernel(x, edge_index, edge_attr, Wn, bn_enc, W, b, We, be, root, gamma, beta):
    row = edge_index[0]
    col = edge_index[1]
    row2 = row.reshape(NW, SN, SG, C)
    col2 = col.reshape(NW, SN, SG, C)

    h = _enc_call(x, Wn, bn_enc.reshape(1, D))

    degp = _deg_call(row.reshape(NW, GD, CD)).reshape(NC, NP)
    deg = degp[0, :N] + degp[1, :N] + 1.0
    dis = deg ** -0.5
    nrm2 = _norm_call(dis, row.reshape(NW, 1, EW), col.reshape(NW, 1, EW))
    degc = deg.reshape(N, 1)

    for l in range(L):
        hx, st = _hx_call(h, W[l], b[l].reshape(1, D), root[l].reshape(1, D), degc)
        ee = _ee_call(edge_attr, We[l], be[l].reshape(1, D))
        aggrp = _edge_call(hx, ee, row2, col2,
                           nrm2.reshape(NW, SN, SG, C))
        h = _post_call(aggrp[0, :N], aggrp[1, :N], st,
                       gamma[l].reshape(1, D), beta[l].reshape(1, D),
                       relu=(l < L - 1))
    return h


# speedup vs baseline: 7.5169x; 1.7709x over previous
import functools

import jax
import jax.numpy as jnp
from jax import lax
from jax.experimental import pallas as pl
from jax.experimental.pallas import tpu as pltpu
from jax.experimental.pallas import tpu_sc as plsc

N = 10000
E = 320000
D = 128
ED = 16
L = 3

NC = 2
NS = 16
NW = NC * NS
EW = E // NW
C = 40
G = EW // C
SG = 10
SN = G // SG
NP = 10240
RS = NP // NS
WB = 32

_MESH = plsc.VectorSubcoreMesh(core_axis_name="c", subcore_axis_name="s")


def _wid():
    return lax.axis_index("s") * NC + lax.axis_index("c")


GD = 125
CD = 80


def _deg_body(row_hbm, out_hbm, deg_sh, row_v, ones_v, buf_v):
    cid = lax.axis_index("c")
    sid = lax.axis_index("s")
    wid = _wid()
    for i in range(CD // 16):
        ones_v[pl.ds(i * 16, 16)] = jnp.ones((16,), jnp.float32)
    for i in range(RS // 16):
        buf_v[pl.ds(i * 16, 16)] = jnp.zeros((16,), jnp.float32)
    pltpu.sync_copy(buf_v, deg_sh.at[pl.ds(sid * RS, RS)])
    plsc.subcore_barrier()
    pltpu.sync_copy(row_hbm.at[wid], row_v)

    def chunk(g, carry):
        pltpu.sync_copy(ones_v, deg_sh.at[row_v.at[g]], add=True)
        return carry

    lax.fori_loop(0, GD, chunk, 0)
    plsc.subcore_barrier()
    pltpu.sync_copy(deg_sh.at[pl.ds(sid * RS, RS)], buf_v)
    pltpu.sync_copy(buf_v, out_hbm.at[pl.ds(cid * NP + sid * RS, RS)])


_deg_call = pl.kernel(
    _deg_body,
    out_type=jax.ShapeDtypeStruct((NC * NP,), jnp.float32),
    mesh=_MESH,
    compiler_params=pltpu.CompilerParams(needs_layout_passes=False),
    scratch_types=[
        pltpu.VMEM_SHARED((NP,), jnp.float32),
        pltpu.VMEM((GD, CD), jnp.int32),
        pltpu.VMEM((CD,), jnp.float32),
        pltpu.VMEM((RS,), jnp.float32),
    ],
)


def _norm_body(dis_hbm, row_hbm, col_hbm, out_hbm, dis_v, row_v, col_v, nrm_v):
    wid = _wid()
    pltpu.sync_copy(dis_hbm, dis_v)
    pltpu.sync_copy(row_hbm.at[wid], row_v)
    pltpu.sync_copy(col_hbm.at[wid], col_v)

    def step(i, carry):
        rv = row_v[0, pl.ds(i * 16, 16)]
        cv = col_v[0, pl.ds(i * 16, 16)]
        a = plsc.load_gather(dis_v, [rv])
        bb = plsc.load_gather(dis_v, [cv])
        nrm_v[0, pl.ds(i * 16, 16)] = a * bb
        return carry

    lax.fori_loop(0, EW // 16, step, 0)
    pltpu.sync_copy(nrm_v, out_hbm.at[wid])


_norm_call = pl.kernel(
    _norm_body,
    out_type=jax.ShapeDtypeStruct((NW, 1, EW), jnp.float32),
    mesh=_MESH,
    compiler_params=pltpu.CompilerParams(needs_layout_passes=False),
    scratch_types=[
        pltpu.VMEM((N,), jnp.float32),
        pltpu.VMEM((1, EW), jnp.int32),
        pltpu.VMEM((1, EW), jnp.int32),
        pltpu.VMEM((1, EW), jnp.float32),
    ],
)


def _edge_body(hx_hbm, ee_hbm, row_hbm, col_hbm, nrm_hbm, out_hbm,
               aggr_sh, row_v, col_v, nrm_v,
               hxg0, hxg1, eev0, eev1, msg0, msg1, sem0, sem1, ssem0, ssem1):
    cid = lax.axis_index("c")
    sid = lax.axis_index("s")
    wid = _wid()

    def zero_block(i, carry):
        for j in range(D // 16):
            msg0[i, pl.ds(j * 16, 16)] = jnp.zeros((16,), jnp.float32)
        return carry

    lax.fori_loop(0, C, zero_block, 0)
    for k in range(RS // C):
        pltpu.sync_copy(msg0, aggr_sh.at[pl.ds(sid * RS + k * C, C)])
    plsc.subcore_barrier()

    ebase = wid * EW
    pad0 = NP - C
    pad1 = NP - 2 * C

    def issue(s, g, hxg, eev, sem):
        pltpu.async_copy(hx_hbm.at[row_v.at[g]], hxg, sem)
        pltpu.async_copy(ee_hbm.at[pl.ds(ebase + (s * SG + g) * C, C)], eev, sem)

    def wait(s, g, hxg, eev, sem):
        pltpu.make_async_copy(hx_hbm.at[row_v.at[g]], hxg, sem).wait()
        pltpu.make_async_copy(
            ee_hbm.at[pl.ds(ebase + (s * SG + g) * C, C)], eev, sem).wait()

    def wait_scatter(msg, ssem):
        pltpu.make_async_copy(msg, aggr_sh.at[pl.ds(0, C)], ssem).wait()

    def compute_chunk(g, hxg, eev, msg):
        g16 = jnp.zeros((16,), jnp.int32) + g

        def edge(e2, e16):
            for u in range(2):
                e = e2 * 2 + u
                nv = plsc.load_gather(nrm_v, [g16, e16 + u])
                for j in range(D // 16):
                    hv = hxg[e, pl.ds(j * 16, 16)]
                    av = eev[e, pl.ds(j * 16, 16)]
                    msg[e, pl.ds(j * 16, 16)] = nv * jnp.maximum(hv + av, 0.0)
            return e16 + 2

        lax.fori_loop(0, C // 2, edge, jnp.zeros((16,), jnp.int32))

    def superchunk(s, carry):
        pltpu.sync_copy(row_hbm.at[wid, s], row_v)
        pltpu.sync_copy(col_hbm.at[wid, s], col_v)
        pltpu.sync_copy(nrm_hbm.at[wid, s], nrm_v)
        issue(s, 0, hxg0, eev0, sem0)
        pltpu.async_copy(msg0, aggr_sh.at[pl.ds(pad0, C)], ssem0)
        pltpu.async_copy(msg1, aggr_sh.at[pl.ds(pad1, C)], ssem1)

        def pair(g2, c1):
            a = 2 * g2
            b = a + 1
            issue(s, b, hxg1, eev1, sem1)
            wait(s, a, hxg0, eev0, sem0)
            wait_scatter(msg0, ssem0)
            compute_chunk(a, hxg0, eev0, msg0)
            pltpu.async_copy(msg0, aggr_sh.at[col_v.at[a]], ssem0, add=True)

            @pl.when(g2 < SG // 2 - 1)
            def _():
                issue(s, a + 2, hxg0, eev0, sem0)

            wait(s, b, hxg1, eev1, sem1)
            wait_scatter(msg1, ssem1)
            compute_chunk(b, hxg1, eev1, msg1)
            pltpu.async_copy(msg1, aggr_sh.at[col_v.at[b]], ssem1, add=True)
            return c1

        lax.fori_loop(0, SG // 2, pair, 0)
        wait_scatter(msg0, ssem0)
        wait_scatter(msg1, ssem1)
        return carry

    lax.fori_loop(0, SN, superchunk, 0)
    plsc.subcore_barrier()
    for k in range(RS // C):
        r0 = sid * RS + k * C
        pltpu.sync_copy(aggr_sh.at[pl.ds(r0, C)], msg0)
        pltpu.sync_copy(msg0, out_hbm.at[cid, pl.ds(r0, C)])


_edge_call = pl.kernel(
    _edge_body,
    out_type=jax.ShapeDtypeStruct((NC, NP, D), jnp.float32),
    mesh=_MESH,
    compiler_params=pltpu.CompilerParams(needs_layout_passes=False),
    scratch_types=[
        pltpu.VMEM_SHARED((NP, D), jnp.float32),
        pltpu.VMEM((SG, C), jnp.int32),
        pltpu.VMEM((SG, C), jnp.int32),
        pltpu.VMEM((SG, C), jnp.float32),
        pltpu.VMEM((C, D), jnp.float32),
        pltpu.VMEM((C, D), jnp.float32),
        pltpu.VMEM((C, D), jnp.float32),
        pltpu.VMEM((C, D), jnp.float32),
        pltpu.VMEM((C, D), jnp.float32),
        pltpu.VMEM((C, D), jnp.float32),
        pltpu.SemaphoreType.DMA,
        pltpu.SemaphoreType.DMA,
        pltpu.SemaphoreType.DMA,
        pltpu.SemaphoreType.DMA,
    ],
)


def _enc_tc(x_ref, w_ref, b_ref, o_ref):
    o_ref[...] = lax.dot_general(
        x_ref[...], w_ref[...], (((1,), (1,)), ((), ())),
        preferred_element_type=jnp.float32) + b_ref[...]


def _hx_tc(h_ref, w_ref, b_ref, r_ref, dg_ref, hx_ref, st_ref):
    hx = lax.dot_general(
        h_ref[...], w_ref[...], (((1,), (1,)), ((), ())),
        preferred_element_type=jnp.float32) + b_ref[...]
    hx_ref[...] = hx
    st_ref[...] = jnp.maximum(hx + r_ref[...], 0.0) / dg_ref[...]


BE = 8000


def _ee_tc(ea_ref, w_ref, b_ref, o_ref):
    o_ref[...] = lax.dot_general(
        ea_ref[...], w_ref[...], (((1,), (1,)), ((), ())),
        preferred_element_type=jnp.float32) + b_ref[...]


def _post_tc(a0_ref, a1_ref, st_ref, g_ref, bt_ref, o_ref, *, relu):
    t = a0_ref[...] + a1_ref[...] + st_ref[...]
    mean = jnp.mean(t, axis=0, keepdims=True)
    var = jnp.mean((t - mean) ** 2, axis=0, keepdims=True)
    y = (t - mean) / jnp.sqrt(var + 1e-5) * g_ref[...] + bt_ref[...]
    o_ref[...] = jnp.maximum(y, 0.0) if relu else y


def _enc_call(x, Wn, bn):
    return pl.pallas_call(
        _enc_tc,
        out_shape=jax.ShapeDtypeStruct((N, D), jnp.float32),
    )(x, Wn, bn)


def _hx_call(h, Wl, bl, rl, degc):
    return pl.pallas_call(
        _hx_tc,
        out_shape=(jax.ShapeDtypeStruct((N, D), jnp.float32),
                   jax.ShapeDtypeStruct((N, D), jnp.float32)),
    )(h, Wl, bl, rl, degc)


def _ee_call(edge_attr, Wel, bel):
    return pl.pallas_call(
        _ee_tc,
        grid=(E // BE,),
        in_specs=[pl.BlockSpec((BE, ED), lambda i: (i, 0)),
                  pl.BlockSpec((D, ED), lambda i: (0, 0)),
                  pl.BlockSpec((1, D), lambda i: (0, 0))],
        out_specs=pl.BlockSpec((BE, D), lambda i: (i, 0)),
        out_shape=jax.ShapeDtypeStruct((E, D), jnp.float32),
    )(edge_attr, Wel, bel)


def _post_call(a0, a1, st, gl, btl, relu):
    return pl.pallas_call(
        functools.partial(_post_tc, relu=relu),
        out_shape=jax.ShapeDtypeStruct((N, D), jnp.float32),
    )(a0, a1, st, gl, btl)


def kernel(x, edge_index, edge_attr, Wn, bn_enc, W, b, We, be, root, gamma, beta):
    row = edge_index[0]
    col = edge_index[1]
    row2 = row.reshape(NW, SN, SG, C)
    col2 = col.reshape(NW, SN, SG, C)

    h = _enc_call(x, Wn, bn_enc.reshape(1, D))

    degp = _deg_call(row.reshape(NW, GD, CD)).reshape(NC, NP)
    deg = degp[0, :N] + degp[1, :N] + 1.0
    dis = deg ** -0.5
    nrm2 = _norm_call(dis, row.reshape(NW, 1, EW), col.reshape(NW, 1, EW))
    degc = deg.reshape(N, 1)

    for l in range(L):
        hx, st = _hx_call(h, W[l], b[l].reshape(1, D), root[l].reshape(1, D), degc)
        ee = _ee_call(edge_attr, We[l], be[l].reshape(1, D))
        aggrp = _edge_call(hx, ee, row2, col2,
                           nrm2.reshape(NW, SN, SG, C))
        h = _post_call(aggrp[0, :N], aggrp[1, :N], st,
                       gamma[l].reshape(1, D), beta[l].reshape(1, D),
                       relu=(l < L - 1))
    return h
